# baseline (speedup 1.0000x reference)
import functools

import numpy as _np

import jax
import jax.numpy as jnp
from jax import lax
from jax.experimental import pallas as pl
from jax.experimental.pallas import tpu as pltpu
from jax.experimental.pallas import tpu_sc as plsc

_BSZ = 64
_BEAM = 4
_VOCAB = 100000
_PAD = 1
_EOS = 2
_CAND = 8
_NT = 782
_NTP = 800
_TPW = 25
_NW = 32
_BPW = _BSZ // _NW
_T0 = 296
_SPW = 16
_NEG = _np.float32(-_np.inf)
_IMAX = _np.int32(2**31 - 1)


def _mesh():
  return plsc.VectorSubcoreMesh(core_axis_name="c", subcore_axis_name="s",
                                num_cores=2, num_subcores=16)


def _wid():
  return lax.axis_index("s") * 2 + lax.axis_index("c")


def _treemax(accs):
  while len(accs) > 1:
    accs = [jnp.maximum(accs[i], accs[i + 1]) if i + 1 < len(accs)
            else accs[i] for i in range(0, len(accs), 2)]
  return accs[0]


def _chainmax(load, n):
  k = min(5, n)
  accs = [load(v) for v in range(k)]
  for v in range(k, n):
    accs[v % k] = jnp.maximum(accs[v % k], load(v))
  return _treemax(accs)


def _bm_body(lpt_hbm, o_bm, buf_a, buf_b, bmv_a, bmv_b, sem_a, sem_b,
             sem_c, sem_d):
  w = _wid()
  t0 = _T0 + w * _SPW
  ntiles = jnp.maximum(0, jnp.minimum(_SPW, _NT - t0))

  def _xfer(buf, ti, sem, start):
    t = t0 + ti

    @pl.when(t == _NT - 1)
    def _():
      cp = pltpu.make_async_copy(lpt_hbm.at[pl.ds(t * 128, 32)],
                                 buf.at[pl.ds(0, 32)], sem)
      cp.start() if start else cp.wait()

    @pl.when(t < _NT - 1)
    def _():
      cp = pltpu.make_async_copy(lpt_hbm.at[pl.ds(t * 128, 128)], buf, sem)
      cp.start() if start else cp.wait()

  def _proc(buf, bmv, ti, semo):
    t = t0 + ti

    @pl.when(t == 0)
    def _():
      def padfix(rg, _):
        buf[1, pl.ds(rg * 16, 16)] = jnp.full((16,), _NEG, jnp.float32)
        return 0
      lax.fori_loop(0, 16, padfix, 0)

    @pl.when(ti >= 2)
    def _():
      pltpu.make_async_copy(bmv, o_bm.at[pl.ds((t - 2) * 256, 256)],
                            semo).wait()

    @pl.when(t == _NT - 1)
    def _():
      def rg_ragged(rg, _):
        acc = _chainmax(lambda v: buf[v, pl.ds(rg * 16, 16)], 32)
        bmv[pl.ds(rg * 16, 16)] = acc
        return 0
      lax.fori_loop(0, 16, rg_ragged, 0)

    @pl.when(t < _NT - 1)
    def _():
      def rg_full(rg, _):
        acc = _chainmax(lambda v: buf[v, pl.ds(rg * 16, 16)], 128)
        bmv[pl.ds(rg * 16, 16)] = acc
        return 0
      lax.fori_loop(0, 16, rg_full, 0)

    pltpu.make_async_copy(bmv, o_bm.at[pl.ds(t * 256, 256)], semo).start()

  @pl.when(ntiles > 0)
  def _():
    _xfer(buf_a, 0, sem_a, True)

  def pipe(i, _):
    ta = 2 * i
    tb = 2 * i + 1

    @pl.when(tb < ntiles)
    def _():
      _xfer(buf_b, tb, sem_b, True)

    @pl.when(ta < ntiles)
    def _():
      _xfer(buf_a, ta, sem_a, False)
      _proc(buf_a, bmv_a, ta, sem_c)

    @pl.when(ta + 2 < ntiles)
    def _():
      _xfer(buf_a, ta + 2, sem_a, True)

    @pl.when(tb < ntiles)
    def _():
      _xfer(buf_b, tb, sem_b, False)
      _proc(buf_b, bmv_b, tb, sem_d)

    return 0

  lax.fori_loop(0, (_SPW + 1) // 2, pipe, 0)

  na = (ntiles + 1) >> 1
  nb = ntiles >> 1

  @pl.when(na > 0)
  def _():
    pltpu.make_async_copy(bmv_a, o_bm.at[pl.ds((t0 + 2 * na - 2) * 256, 256)],
                          sem_c).wait()

  @pl.when(nb > 0)
  def _():
    pltpu.make_async_copy(bmv_b, o_bm.at[pl.ds((t0 + 2 * nb - 1) * 256, 256)],
                          sem_d).wait()

  @pl.when(w == _NW - 1)
  def _():
    def negfill(rg, _):
      bmv_a[pl.ds(rg * 16, 16)] = jnp.full((16,), _NEG, jnp.float32)
      return 0
    lax.fori_loop(0, 16, negfill, 0)

    def vtile(q, _):
      pltpu.sync_copy(bmv_a, o_bm.at[pl.ds((_NT + q) * 256, 256)])
      return 0
    lax.fori_loop(0, _NTP - _NT, vtile, 0)


def _sel_body(bmt_hbm, sc_hbm, o_sel, bmv2, scores_v, st_sel, iv_st):
  u = _wid()
  lanes = lax.iota(jnp.int32, 16)
  lane0 = lanes == 0

  def _sget(ref, idx):
    return plsc.load_gather(ref, [jnp.full((16,), idx, jnp.int32)])[0]

  def _sput(ref, idx, val):
    plsc.store_scatter(ref, [jnp.full((16,), idx, jnp.int32)],
                       jnp.full((16,), val), mask=lane0)

  pltpu.sync_copy(bmt_hbm.at[pl.ds(8 * u, 8)], bmv2)
  pltpu.sync_copy(sc_hbm, scores_v)

  _GV = 10
  _GPB = (_NTP // 16) // _GV

  def _grp(rl, q, beam, off):
    return _chainmax(
        lambda v: bmv2[rl, pl.ds((q * _GV + v) * 16, 16)] + off, _GV)

  def per_batch(b2, _):
    batch = u * _BPW + b2
    offs = [_sget(scores_v, batch * _BEAM + beam) for beam in range(_BEAM)]
    sup = []
    for beam in range(_BEAM):
      rl = b2 * _BEAM + beam
      for q in range(_GPB):
        sup.append(_grp(rl, q, beam, offs[beam]))

    def select(k, sup):
      mm = jnp.max(_treemax(list(sup)))
      gv = jnp.full((16,), _IMAX, jnp.int32)
      for gi in range(_BEAM * _GPB):
        gv = jnp.minimum(gv, jnp.where(sup[gi] == mm, jnp.int32(gi), _IMAX))
      gstar = jnp.min(gv)
      iv = jnp.full((16,), _IMAX, jnp.int32)
      for gi in range(_BEAM * _GPB):
        beam = gi // _GPB
        q = gi - beam * _GPB
        rl = b2 * _BEAM + beam

        @pl.when(gstar == gi)
        def _(beam=beam, q=q, rl=rl):
          ivl = jnp.full((16,), _IMAX, jnp.int32)
          for v in range(_GV):
            x = bmv2[rl, pl.ds((q * _GV + v) * 16, 16)] + offs[beam]
            ivl = jnp.minimum(
                ivl,
                jnp.where(x == mm,
                          beam * 1024 + (q * _GV + v) * 16 + lanes, _IMAX))
          iv_st[...] = ivl

      gb = jnp.min(iv_st[...])
      _sput(st_sel, b2 * 8 + k, gb)
      beam_g = gb >> 10
      t_g = gb & 1023
      plsc.store_scatter(bmv2,
                         [jnp.full((16,), b2 * _BEAM + beam_g, jnp.int32),
                          jnp.full((16,), t_g, jnp.int32)],
                         jnp.full((16,), _NEG, jnp.float32), mask=lane0)
      nsup = []
      for gi in range(_BEAM * _GPB):
        beam = gi // _GPB
        q = gi - beam * _GPB
        rl = b2 * _BEAM + beam
        red = _grp(rl, q, beam, offs[beam])
        nsup.append(jnp.where(gstar == gi, red, sup[gi]))
      return tuple(nsup)

    lax.fori_loop(0, _CAND, select, tuple(sup))
    return 0

  lax.fori_loop(0, _BPW, per_batch, 0)
  pltpu.sync_copy(st_sel, o_sel.at[pl.ds(16 * u, 16)])


def _ex_body(cand_hbm, sel_hbm, sc_hbm,
             o_cs, o_ci, o_cb, o_as, o_ai, o_ab,
             cbuf, selv, scores_v,
             st_cs, st_ci, st_cb, st_as, st_ai, st_ab):
  u = _wid()
  lanes = lax.iota(jnp.int32, 16)
  lane0 = lanes == 0

  def _sget(ref, idx):
    return plsc.load_gather(ref, [jnp.full((16,), idx, jnp.int32)])[0]

  def _sput(ref, idx, val):
    plsc.store_scatter(ref, [jnp.full((16,), idx, jnp.int32)],
                       jnp.full((16,), val), mask=lane0)

  def _splat(x):
    return jnp.full((16,), x, jnp.int32)

  pltpu.sync_copy(sel_hbm.at[pl.ds(16 * u, 16)], selv)
  pltpu.sync_copy(sc_hbm, scores_v)

  for b2 in range(_BPW):
    batch = u * _BPW + b2
    pltpu.sync_copy(cand_hbm.at[batch], cbuf)

    offs = []
    gbases = []
    for s in range(_CAND):
      gb_s = _sget(selv, b2 * 8 + s)
      beam_s = gb_s >> 10
      t_s = gb_s & 1023
      offs.append(_sget(scores_v, batch * _BEAM + beam_s))
      gbases.append(beam_s * _VOCAB + t_s * 128)

      @pl.when(t_s == 0)
      def _(s=s):
        plsc.store_scatter(cbuf, [_splat(s), _splat(_PAD)],
                           jnp.full((16,), _NEG, jnp.float32), mask=lane0)

      @pl.when(t_s == _NT - 1)
      def _(s=s):
        for v in range(2, 8):
          plsc.store_scatter(cbuf, [_splat(s), v * 16 + lanes],
                             jnp.full((16,), _NEG, jnp.float32))

    def extract(k, _):
      m = jnp.full((16,), _NEG, jnp.float32)
      for s in range(_CAND):
        m = jnp.maximum(
            m, _chainmax(lambda v, s=s: cbuf[s, pl.ds(v * 16, 16)] + offs[s],
                         8))
      mm = jnp.max(m)
      iv = jnp.full((16,), _IMAX, jnp.int32)
      for s in range(_CAND):
        for v in range(8):
          x = cbuf[s, pl.ds(v * 16, 16)] + offs[s]
          iv = jnp.minimum(
              iv, jnp.where(x == mm, gbases[s] + v * 16 + lanes, _IMAX))
      g = jnp.min(iv)
      for s in range(_CAND):
        pos = g - gbases[s]

        @pl.when(jnp.logical_and(pos >= 0, pos < 128))
        def _(s=s, pos=pos):
          plsc.store_scatter(cbuf, [_splat(s), _splat(pos)],
                             jnp.full((16,), _NEG, jnp.float32), mask=lane0)

      beam_k = ((g >= _VOCAB).astype(jnp.int32)
                + (g >= 2 * _VOCAB).astype(jnp.int32)
                + (g >= 3 * _VOCAB).astype(jnp.int32))
      _sput(st_cs, b2 * 8 + k, mm)
      _sput(st_ci, b2 * 8 + k, g - beam_k * _VOCAB)
      _sput(st_cb, b2 * 8 + k, beam_k)
      return 0

    lax.fori_loop(0, _CAND, extract, 0)

  tok_vec = st_ci[...]
  is_eos = (tok_vec == _EOS).astype(jnp.int32)
  k_l = lanes & 7
  b2_l = lanes >> 3
  key = b2_l * 100 + is_eos * 8 + k_l
  hyp = plsc.sort_key_val(key, lanes)
  if isinstance(hyp, (list, tuple)):
    hyp = hyp[-1]
  valid = k_l < _BEAM
  as_full = plsc.load_gather(st_cs, [hyp])
  ai_full = plsc.load_gather(st_ci, [hyp])
  ab_full = plsc.load_gather(st_cb, [hyp])
  batch_vec = (u * _BPW + b2_l) * _BEAM
  st_as[...] = jnp.where(valid, as_full, jnp.float32(0.0))
  st_ai[...] = jnp.where(valid, ai_full, jnp.int32(0))
  st_ab[...] = jnp.where(valid, ab_full + batch_vec, jnp.int32(0))

  base = 16 * u
  pltpu.sync_copy(st_cs, o_cs.at[pl.ds(base, 16)])
  pltpu.sync_copy(st_ci, o_ci.at[pl.ds(base, 16)])
  pltpu.sync_copy(st_cb, o_cb.at[pl.ds(base, 16)])
  pltpu.sync_copy(st_as, o_as.at[pl.ds(base, 16)])
  pltpu.sync_copy(st_ai, o_ai.at[pl.ds(base, 16)])
  pltpu.sync_copy(st_ab, o_ab.at[pl.ds(base, 16)])


def _launch_bm():
  return functools.partial(
      pl.kernel, mesh=_mesh(),
      out_type=[jax.ShapeDtypeStruct((_NTP * 256,), jnp.float32)],
      scratch_types=[
          pltpu.VMEM((128, 256), jnp.float32),
          pltpu.VMEM((128, 256), jnp.float32),
          pltpu.VMEM((256,), jnp.float32),
          pltpu.VMEM((256,), jnp.float32),
          pltpu.SemaphoreType.DMA,
          pltpu.SemaphoreType.DMA,
          pltpu.SemaphoreType.DMA,
          pltpu.SemaphoreType.DMA,
      ],
      compiler_params=pltpu.CompilerParams(needs_layout_passes=False),
  )(_bm_body)


def _launch_sel():
  return functools.partial(
      pl.kernel, mesh=_mesh(),
      out_type=[jax.ShapeDtypeStruct((_BSZ * 8,), jnp.int32)],
      scratch_types=[
          pltpu.VMEM((8, _NTP), jnp.float32),
          pltpu.VMEM((256,), jnp.float32),
          pltpu.VMEM((16,), jnp.int32),
          pltpu.VMEM((16,), jnp.int32),
      ],
      compiler_params=pltpu.CompilerParams(needs_layout_passes=False),
  )(_sel_body)


def _launch_ex():
  return functools.partial(
      pl.kernel, mesh=_mesh(),
      out_type=[
          jax.ShapeDtypeStruct((_BSZ * 8,), jnp.float32),
          jax.ShapeDtypeStruct((_BSZ * 8,), jnp.int32),
          jax.ShapeDtypeStruct((_BSZ * 8,), jnp.int32),
          jax.ShapeDtypeStruct((_BSZ * 8,), jnp.float32),
          jax.ShapeDtypeStruct((_BSZ * 8,), jnp.int32),
          jax.ShapeDtypeStruct((_BSZ * 8,), jnp.int32),
      ],
      scratch_types=[
          pltpu.VMEM((_CAND, 128), jnp.float32),
          pltpu.VMEM((16,), jnp.int32),
          pltpu.VMEM((256,), jnp.float32),
          pltpu.VMEM((16,), jnp.float32),
          pltpu.VMEM((16,), jnp.int32),
          pltpu.VMEM((16,), jnp.int32),
          pltpu.VMEM((16,), jnp.float32),
          pltpu.VMEM((16,), jnp.int32),
          pltpu.VMEM((16,), jnp.int32),
      ],
      compiler_params=pltpu.CompilerParams(needs_layout_passes=False),
  )(_ex_body)


def _tc_body(lpt_ref, o_ref):
  g = pl.program_id(0)
  x = lpt_ref[...]
  row = jax.lax.broadcasted_iota(jnp.int32, (1024, 256), 0)
  x = jnp.where(jnp.logical_and(g == 0, row == _PAD), _NEG, x)
  o_ref[...] = jnp.max(x.reshape(8, 128, 256), axis=1)


def _launch_tc():
  return pl.pallas_call(
      _tc_body,
      grid=(_T0 // 8,),
      in_specs=[pl.BlockSpec((1024, 256), lambda g: (g, 0))],
      out_specs=pl.BlockSpec((8, 256), lambda g: (g, 0)),
      out_shape=jax.ShapeDtypeStruct((_T0, 256), jnp.float32),
  )


@jax.jit
def kernel(lprobs, scores_prev):
  lpt = lprobs.T
  bm_tc = _launch_tc()(lpt)
  (bm1d,) = _launch_bm()(lpt)
  bm_sc = bm1d.reshape(_NTP, 256)
  bmt = jnp.concatenate([bm_tc, bm_sc[_T0:]], axis=0).T
  (sel,) = _launch_sel()(bmt, scores_prev)
  sel64 = sel.reshape(_BSZ, 8)
  beam = sel64 >> 10
  t = sel64 & 1023
  rows = jnp.arange(_BSZ, dtype=jnp.int32)[:, None] * _BEAM + beam
  cols = t[..., None] * 128 + jnp.arange(128, dtype=jnp.int32)[None, None, :]
  cols = jnp.minimum(cols, _VOCAB - 1)
  cand = lprobs[rows[:, :, None], cols]
  cs, ci, cb, as_, ai, ab = _launch_ex()(cand, sel, scores_prev)
  return (cs.reshape(_BSZ, 8), ci.reshape(_BSZ, 8), cb.reshape(_BSZ, 8),
          as_.reshape(_BSZ, 8)[:, :_BEAM], ai.reshape(_BSZ, 8)[:, :_BEAM],
          ab.reshape(_BSZ, 8)[:, :_BEAM])


# speedup vs baseline: 1.0418x; 1.0418x over previous
import functools

import numpy as _np

import jax
import jax.numpy as jnp
from jax import lax
from jax.experimental import pallas as pl
from jax.experimental.pallas import tpu as pltpu
from jax.experimental.pallas import tpu_sc as plsc

_BSZ = 64
_BEAM = 4
_VOCAB = 100000
_PAD = 1
_EOS = 2
_CAND = 8
_NT = 782
_NTP = 800
_TPW = 25
_NW = 32
_BPW = _BSZ // _NW
_T0 = 344
_SPW = 14
_NEG = _np.float32(-_np.inf)
_IMAX = _np.int32(2**31 - 1)


def _mesh():
  return plsc.VectorSubcoreMesh(core_axis_name="c", subcore_axis_name="s",
                                num_cores=2, num_subcores=16)


def _wid():
  return lax.axis_index("s") * 2 + lax.axis_index("c")


def _treemax(accs):
  while len(accs) > 1:
    accs = [jnp.maximum(accs[i], accs[i + 1]) if i + 1 < len(accs)
            else accs[i] for i in range(0, len(accs), 2)]
  return accs[0]


def _chainmax(load, n):
  k = min(5, n)
  accs = [load(v) for v in range(k)]
  for v in range(k, n):
    accs[v % k] = jnp.maximum(accs[v % k], load(v))
  return _treemax(accs)


def _bm_body(lpt_hbm, o_bm, buf_a, buf_b, bmv_a, bmv_b, sem_a, sem_b,
             sem_c, sem_d):
  w = _wid()
  t0 = _T0 + w * _SPW
  ntiles = jnp.maximum(0, jnp.minimum(_SPW, _NT - t0))

  def _xfer(buf, ti, sem, start):
    t = t0 + ti

    @pl.when(t == _NT - 1)
    def _():
      cp = pltpu.make_async_copy(lpt_hbm.at[pl.ds(t * 128, 32)],
                                 buf.at[pl.ds(0, 32)], sem)
      cp.start() if start else cp.wait()

    @pl.when(t < _NT - 1)
    def _():
      cp = pltpu.make_async_copy(lpt_hbm.at[pl.ds(t * 128, 128)], buf, sem)
      cp.start() if start else cp.wait()

  def _proc(buf, bmv, ti, semo):
    t = t0 + ti

    @pl.when(t == 0)
    def _():
      def padfix(rg, _):
        buf[1, pl.ds(rg * 16, 16)] = jnp.full((16,), _NEG, jnp.float32)
        return 0
      lax.fori_loop(0, 16, padfix, 0)

    @pl.when(ti >= 2)
    def _():
      pltpu.make_async_copy(bmv, o_bm.at[pl.ds((t - 2) * 256, 256)],
                            semo).wait()

    @pl.when(t == _NT - 1)
    def _():
      def rg_ragged(rg, _):
        acc = _chainmax(lambda v: buf[v, pl.ds(rg * 16, 16)], 32)
        bmv[pl.ds(rg * 16, 16)] = acc
        return 0
      lax.fori_loop(0, 16, rg_ragged, 0)

    @pl.when(t < _NT - 1)
    def _():
      def rg_full(rg, _):
        acc = _chainmax(lambda v: buf[v, pl.ds(rg * 16, 16)], 128)
        bmv[pl.ds(rg * 16, 16)] = acc
        return 0
      lax.fori_loop(0, 16, rg_full, 0)

    pltpu.make_async_copy(bmv, o_bm.at[pl.ds(t * 256, 256)], semo).start()

  @pl.when(ntiles > 0)
  def _():
    _xfer(buf_a, 0, sem_a, True)

  def pipe(i, _):
    ta = 2 * i
    tb = 2 * i + 1

    @pl.when(tb < ntiles)
    def _():
      _xfer(buf_b, tb, sem_b, True)

    @pl.when(ta < ntiles)
    def _():
      _xfer(buf_a, ta, sem_a, False)
      _proc(buf_a, bmv_a, ta, sem_c)

    @pl.when(ta + 2 < ntiles)
    def _():
      _xfer(buf_a, ta + 2, sem_a, True)

    @pl.when(tb < ntiles)
    def _():
      _xfer(buf_b, tb, sem_b, False)
      _proc(buf_b, bmv_b, tb, sem_d)

    return 0

  lax.fori_loop(0, (_SPW + 1) // 2, pipe, 0)

  na = (ntiles + 1) >> 1
  nb = ntiles >> 1

  @pl.when(na > 0)
  def _():
    pltpu.make_async_copy(bmv_a, o_bm.at[pl.ds((t0 + 2 * na - 2) * 256, 256)],
                          sem_c).wait()

  @pl.when(nb > 0)
  def _():
    pltpu.make_async_copy(bmv_b, o_bm.at[pl.ds((t0 + 2 * nb - 1) * 256, 256)],
                          sem_d).wait()

  @pl.when(w == _NW - 1)
  def _():
    def negfill(rg, _):
      bmv_a[pl.ds(rg * 16, 16)] = jnp.full((16,), _NEG, jnp.float32)
      return 0
    lax.fori_loop(0, 16, negfill, 0)

    def vtile(q, _):
      pltpu.sync_copy(bmv_a, o_bm.at[pl.ds((_NT + q) * 256, 256)])
      return 0
    lax.fori_loop(0, _NTP - _NT, vtile, 0)


def _sel_body(bmt_hbm, sc_hbm, o_sel, bmv2, scores_v, st_sel, iv_st):
  u = _wid()
  lanes = lax.iota(jnp.int32, 16)
  lane0 = lanes == 0

  def _sget(ref, idx):
    return plsc.load_gather(ref, [jnp.full((16,), idx, jnp.int32)])[0]

  def _sput(ref, idx, val):
    plsc.store_scatter(ref, [jnp.full((16,), idx, jnp.int32)],
                       jnp.full((16,), val), mask=lane0)

  pltpu.sync_copy(bmt_hbm.at[pl.ds(8 * u, 8)], bmv2)
  pltpu.sync_copy(sc_hbm, scores_v)

  _GV = 10
  _GPB = (_NTP // 16) // _GV

  def _grp(rl, q, beam, off):
    return _chainmax(
        lambda v: bmv2[rl, pl.ds((q * _GV + v) * 16, 16)] + off, _GV)

  def per_batch(b2, _):
    batch = u * _BPW + b2
    offs = [_sget(scores_v, batch * _BEAM + beam) for beam in range(_BEAM)]
    sup = []
    for beam in range(_BEAM):
      rl = b2 * _BEAM + beam
      for q in range(_GPB):
        sup.append(_grp(rl, q, beam, offs[beam]))

    def select(k, sup):
      mm = jnp.max(_treemax(list(sup)))
      gv = jnp.full((16,), _IMAX, jnp.int32)
      for gi in range(_BEAM * _GPB):
        gv = jnp.minimum(gv, jnp.where(sup[gi] == mm, jnp.int32(gi), _IMAX))
      gstar = jnp.min(gv)
      iv = jnp.full((16,), _IMAX, jnp.int32)
      for gi in range(_BEAM * _GPB):
        beam = gi // _GPB
        q = gi - beam * _GPB
        rl = b2 * _BEAM + beam

        @pl.when(gstar == gi)
        def _(beam=beam, q=q, rl=rl):
          ivl = jnp.full((16,), _IMAX, jnp.int32)
          for v in range(_GV):
            x = bmv2[rl, pl.ds((q * _GV + v) * 16, 16)] + offs[beam]
            ivl = jnp.minimum(
                ivl,
                jnp.where(x == mm,
                          beam * 1024 + (q * _GV + v) * 16 + lanes, _IMAX))
          iv_st[...] = ivl

      gb = jnp.min(iv_st[...])
      _sput(st_sel, b2 * 8 + k, gb)
      beam_g = gb >> 10
      t_g = gb & 1023
      plsc.store_scatter(bmv2,
                         [jnp.full((16,), b2 * _BEAM + beam_g, jnp.int32),
                          jnp.full((16,), t_g, jnp.int32)],
                         jnp.full((16,), _NEG, jnp.float32), mask=lane0)
      nsup = []
      for gi in range(_BEAM * _GPB):
        beam = gi // _GPB
        q = gi - beam * _GPB
        rl = b2 * _BEAM + beam
        red = _grp(rl, q, beam, offs[beam])
        nsup.append(jnp.where(gstar == gi, red, sup[gi]))
      return tuple(nsup)

    lax.fori_loop(0, _CAND, select, tuple(sup))
    return 0

  lax.fori_loop(0, _BPW, per_batch, 0)
  pltpu.sync_copy(st_sel, o_sel.at[pl.ds(16 * u, 16)])


def _ex_body(cand_hbm, sel_hbm, sc_hbm,
             o_cs, o_ci, o_cb, o_as, o_ai, o_ab,
             cbuf, selv, scores_v,
             st_cs, st_ci, st_cb, st_as, st_ai, st_ab):
  u = _wid()
  lanes = lax.iota(jnp.int32, 16)
  lane0 = lanes == 0

  def _sget(ref, idx):
    return plsc.load_gather(ref, [jnp.full((16,), idx, jnp.int32)])[0]

  def _sput(ref, idx, val):
    plsc.store_scatter(ref, [jnp.full((16,), idx, jnp.int32)],
                       jnp.full((16,), val), mask=lane0)

  def _splat(x):
    return jnp.full((16,), x, jnp.int32)

  pltpu.sync_copy(sel_hbm.at[pl.ds(16 * u, 16)], selv)
  pltpu.sync_copy(sc_hbm, scores_v)

  for b2 in range(_BPW):
    batch = u * _BPW + b2
    pltpu.sync_copy(cand_hbm.at[batch], cbuf)

    offs = []
    gbases = []
    for s in range(_CAND):
      gb_s = _sget(selv, b2 * 8 + s)
      beam_s = gb_s >> 10
      t_s = gb_s & 1023
      offs.append(_sget(scores_v, batch * _BEAM + beam_s))
      gbases.append(beam_s * _VOCAB + t_s * 128)

      @pl.when(t_s == 0)
      def _(s=s):
        plsc.store_scatter(cbuf, [_splat(s), _splat(_PAD)],
                           jnp.full((16,), _NEG, jnp.float32), mask=lane0)

      @pl.when(t_s == _NT - 1)
      def _(s=s):
        for v in range(2, 8):
          plsc.store_scatter(cbuf, [_splat(s), v * 16 + lanes],
                             jnp.full((16,), _NEG, jnp.float32))

    def extract(k, _):
      m = jnp.full((16,), _NEG, jnp.float32)
      for s in range(_CAND):
        m = jnp.maximum(
            m, _chainmax(lambda v, s=s: cbuf[s, pl.ds(v * 16, 16)] + offs[s],
                         8))
      mm = jnp.max(m)
      iv = jnp.full((16,), _IMAX, jnp.int32)
      for s in range(_CAND):
        for v in range(8):
          x = cbuf[s, pl.ds(v * 16, 16)] + offs[s]
          iv = jnp.minimum(
              iv, jnp.where(x == mm, gbases[s] + v * 16 + lanes, _IMAX))
      g = jnp.min(iv)
      for s in range(_CAND):
        pos = g - gbases[s]

        @pl.when(jnp.logical_and(pos >= 0, pos < 128))
        def _(s=s, pos=pos):
          plsc.store_scatter(cbuf, [_splat(s), _splat(pos)],
                             jnp.full((16,), _NEG, jnp.float32), mask=lane0)

      beam_k = ((g >= _VOCAB).astype(jnp.int32)
                + (g >= 2 * _VOCAB).astype(jnp.int32)
                + (g >= 3 * _VOCAB).astype(jnp.int32))
      _sput(st_cs, b2 * 8 + k, mm)
      _sput(st_ci, b2 * 8 + k, g - beam_k * _VOCAB)
      _sput(st_cb, b2 * 8 + k, beam_k)
      return 0

    lax.fori_loop(0, _CAND, extract, 0)

  tok_vec = st_ci[...]
  is_eos = (tok_vec == _EOS).astype(jnp.int32)
  k_l = lanes & 7
  b2_l = lanes >> 3
  key = b2_l * 100 + is_eos * 8 + k_l
  hyp = plsc.sort_key_val(key, lanes)
  if isinstance(hyp, (list, tuple)):
    hyp = hyp[-1]
  valid = k_l < _BEAM
  as_full = plsc.load_gather(st_cs, [hyp])
  ai_full = plsc.load_gather(st_ci, [hyp])
  ab_full = plsc.load_gather(st_cb, [hyp])
  batch_vec = (u * _BPW + b2_l) * _BEAM
  st_as[...] = jnp.where(valid, as_full, jnp.float32(0.0))
  st_ai[...] = jnp.where(valid, ai_full, jnp.int32(0))
  st_ab[...] = jnp.where(valid, ab_full + batch_vec, jnp.int32(0))

  base = 16 * u
  pltpu.sync_copy(st_cs, o_cs.at[pl.ds(base, 16)])
  pltpu.sync_copy(st_ci, o_ci.at[pl.ds(base, 16)])
  pltpu.sync_copy(st_cb, o_cb.at[pl.ds(base, 16)])
  pltpu.sync_copy(st_as, o_as.at[pl.ds(base, 16)])
  pltpu.sync_copy(st_ai, o_ai.at[pl.ds(base, 16)])
  pltpu.sync_copy(st_ab, o_ab.at[pl.ds(base, 16)])


def _launch_bm():
  return functools.partial(
      pl.kernel, mesh=_mesh(),
      out_type=[jax.ShapeDtypeStruct((_NTP * 256,), jnp.float32)],
      scratch_types=[
          pltpu.VMEM((128, 256), jnp.float32),
          pltpu.VMEM((128, 256), jnp.float32),
          pltpu.VMEM((256,), jnp.float32),
          pltpu.VMEM((256,), jnp.float32),
          pltpu.SemaphoreType.DMA,
          pltpu.SemaphoreType.DMA,
          pltpu.SemaphoreType.DMA,
          pltpu.SemaphoreType.DMA,
      ],
      compiler_params=pltpu.CompilerParams(needs_layout_passes=False),
  )(_bm_body)


def _launch_sel():
  return functools.partial(
      pl.kernel, mesh=_mesh(),
      out_type=[jax.ShapeDtypeStruct((_BSZ * 8,), jnp.int32)],
      scratch_types=[
          pltpu.VMEM((8, _NTP), jnp.float32),
          pltpu.VMEM((256,), jnp.float32),
          pltpu.VMEM((16,), jnp.int32),
          pltpu.VMEM((16,), jnp.int32),
      ],
      compiler_params=pltpu.CompilerParams(needs_layout_passes=False),
  )(_sel_body)


def _launch_ex():
  return functools.partial(
      pl.kernel, mesh=_mesh(),
      out_type=[
          jax.ShapeDtypeStruct((_BSZ * 8,), jnp.float32),
          jax.ShapeDtypeStruct((_BSZ * 8,), jnp.int32),
          jax.ShapeDtypeStruct((_BSZ * 8,), jnp.int32),
          jax.ShapeDtypeStruct((_BSZ * 8,), jnp.float32),
          jax.ShapeDtypeStruct((_BSZ * 8,), jnp.int32),
          jax.ShapeDtypeStruct((_BSZ * 8,), jnp.int32),
      ],
      scratch_types=[
          pltpu.VMEM((_CAND, 128), jnp.float32),
          pltpu.VMEM((16,), jnp.int32),
          pltpu.VMEM((256,), jnp.float32),
          pltpu.VMEM((16,), jnp.float32),
          pltpu.VMEM((16,), jnp.int32),
          pltpu.VMEM((16,), jnp.int32),
          pltpu.VMEM((16,), jnp.float32),
          pltpu.VMEM((16,), jnp.int32),
          pltpu.VMEM((16,), jnp.int32),
      ],
      compiler_params=pltpu.CompilerParams(needs_layout_passes=False),
  )(_ex_body)


def _tc_body(lpt_ref, o_ref):
  g = pl.program_id(0)
  x = lpt_ref[...]
  row = jax.lax.broadcasted_iota(jnp.int32, (1024, 256), 0)
  x = jnp.where(jnp.logical_and(g == 0, row == _PAD), _NEG, x)
  o_ref[...] = jnp.max(x.reshape(8, 128, 256), axis=1)


def _launch_tc():
  return pl.pallas_call(
      _tc_body,
      grid=(_T0 // 8,),
      in_specs=[pl.BlockSpec((1024, 256), lambda g: (g, 0))],
      out_specs=pl.BlockSpec((8, 256), lambda g: (g, 0)),
      out_shape=jax.ShapeDtypeStruct((_T0, 256), jnp.float32),
  )


@jax.jit
def kernel(lprobs, scores_prev):
  lpt = lprobs.T
  bm_tc = _launch_tc()(lpt)
  (bm1d,) = _launch_bm()(lpt)
  bm_sc = bm1d.reshape(_NTP, 256)
  bmt = jnp.concatenate([bm_tc, bm_sc[_T0:]], axis=0).T
  (sel,) = _launch_sel()(bmt, scores_prev)
  sel64 = sel.reshape(_BSZ, 8)
  beam = sel64 >> 10
  t = sel64 & 1023
  rows = jnp.arange(_BSZ, dtype=jnp.int32)[:, None] * _BEAM + beam
  cols = t[..., None] * 128 + jnp.arange(128, dtype=jnp.int32)[None, None, :]
  cols = jnp.minimum(cols, _VOCAB - 1)
  cand = lprobs[rows[:, :, None], cols]
  cs, ci, cb, as_, ai, ab = _launch_ex()(cand, sel, scores_prev)
  return (cs.reshape(_BSZ, 8), ci.reshape(_BSZ, 8), cb.reshape(_BSZ, 8),
          as_.reshape(_BSZ, 8)[:, :_BEAM], ai.reshape(_BSZ, 8)[:, :_BEAM],
          ab.reshape(_BSZ, 8)[:, :_BEAM])
